# baseline (speedup 1.0000x reference)
import functools

import jax
import jax.numpy as jnp
from jax import lax
from jax.experimental import pallas as pl
from jax.experimental.pallas import tpu as pltpu
from jax.experimental.pallas import tpu_sc as plsc

NC = 2
NS = 16
NW = NC * NS
LANES = 16

B = 4
S = 8192
D = 768
R = 32
P_PER_W = S // NW
C = P_PER_W // R
C2 = C // 2
DV = D // LANES


def _emb_kernel(ids_hbm, tok_table_hbm, pos_hbm, out_hbm,
                idx_v, pos_v, tok_v,
                sem_g0, sem_g1, sem_st0, sem_st1, sem_p0, sem_p1):
    wid = lax.axis_index("s") * NC + lax.axis_index("c")
    p0 = wid * P_PER_W
    sem_g = (sem_g0, sem_g1)
    sem_st = (sem_st0, sem_st1)
    sem_p = (sem_p0, sem_p1)

    def g_start(b, c, tb):
        pltpu.async_copy(
            tok_table_hbm.at[idx_v.at[b, c]], tok_v.at[tb], sem_g[tb])

    def g_wait(tb):
        pltpu.make_async_copy(
            tok_table_hbm.at[idx_v.at[0, 0]], tok_v.at[tb],
            sem_g[tb]).wait()

    def st_start(b, c, tb):
        pltpu.async_copy(
            tok_v.at[tb], out_hbm.at[pl.ds(b * S + p0 + c * R, R)],
            sem_st[tb])

    def st_wait(tb):
        pltpu.make_async_copy(
            tok_v.at[tb], out_hbm.at[pl.ds(p0, R)], sem_st[tb]).wait()

    def p_start(c, pb):
        pltpu.async_copy(
            pos_hbm.at[pl.ds(p0 + c * R, R)], pos_v.at[pb], sem_p[pb])

    def p_wait(pb):
        pltpu.make_async_copy(
            pos_hbm.at[pl.ds(p0, R)], pos_v.at[pb], sem_p[pb]).wait()

    def add_rows(tb, pb):
        tv = tok_v.at[tb]
        pv = pos_v.at[pb]

        def row_body(r, _):
            for k in range(DV):
                sl = pl.ds(k * LANES, LANES)
                plsc.addupdate(tv.at[r, sl], pv[r, sl])
            return 0

        lax.fori_loop(0, R, row_body, 0, unroll=False)

    pltpu.sync_copy(ids_hbm.at[wid], idx_v)
    p_start(0, 0)
    g_start(0, 0, 0)

    def pair_body(c2, _):
        for rr in range(8):
            b = rr % 4
            tb = rr % 2
            pb = rr // 4
            c = 2 * c2 + pb
            if b == 0:
                p_wait(pb)
                if rr == 0:
                    p_start(c + 1, 1 - pb)
                else:
                    @pl.when(c2 < C2 - 1)
                    def _():
                        p_start(c + 1, 1 - pb)
            if rr == 0:
                @pl.when(c2 > 0)
                def _():
                    st_wait(1 - tb)
                g_start(b + 1, c, 1 - tb)
            elif rr < 7:
                st_wait(1 - tb)
                g_start((rr + 1) % 4, 2 * c2 + (rr + 1) // 4, 1 - tb)
            else:
                @pl.when(c2 < C2 - 1)
                def _():
                    st_wait(1 - tb)
                    g_start(0, 2 * c2 + 2, 1 - tb)
            g_wait(tb)
            add_rows(tb, pb)
            st_start(b, c, tb)
        return 0

    lax.fori_loop(0, C2, pair_body, 0, unroll=False)
    st_wait(0)
    st_wait(1)


@jax.jit
def _emb(ids, token_table, position_table):
    mesh = plsc.VectorSubcoreMesh(
        core_axis_name="c", subcore_axis_name="s", num_cores=NC,
        num_subcores=NS)
    run = pl.kernel(
        _emb_kernel,
        out_type=jax.ShapeDtypeStruct((B * S, D), jnp.float32),
        mesh=mesh,
        scratch_types=[
            pltpu.VMEM((B, C, R), jnp.int32),
            pltpu.VMEM((2, R, D), jnp.float32),
            pltpu.VMEM((2, R, D), jnp.float32),
            pltpu.SemaphoreType.DMA,
            pltpu.SemaphoreType.DMA,
            pltpu.SemaphoreType.DMA,
            pltpu.SemaphoreType.DMA,
            pltpu.SemaphoreType.DMA,
            pltpu.SemaphoreType.DMA,
        ],
    )
    return run(ids, token_table, position_table)


def kernel(token_ids, token_table, position_table):
    ids = token_ids.astype(jnp.int32)
    ids_w = jnp.transpose(ids.reshape(B, NW, C * R), (1, 0, 2))
    ids_w = ids_w.reshape(NW, B, C, R)
    out = _emb(ids_w, token_table, position_table)
    return out.reshape(B, S, D)


# speedup vs baseline: 1.1927x; 1.1927x over previous
import functools

import jax
import jax.numpy as jnp
from jax import lax
from jax.experimental import pallas as pl
from jax.experimental.pallas import tpu as pltpu
from jax.experimental.pallas import tpu_sc as plsc

NC = 2
NS = 16
NW = NC * NS
LANES = 16

B = 4
S = 8192
D = 768
R = 32
P_PER_W = S // NW
C = P_PER_W // R
C2 = C // 2
DV = D // LANES


def _emb_kernel(ids_hbm, tok_table_hbm, pos_hbm, out_hbm,
                idx_v, pos_v, tok_v,
                sem_g0, sem_g1, sem_st0, sem_st1, sem_p0, sem_p1):
    wid = lax.axis_index("s") * NC + lax.axis_index("c")
    p0 = wid * P_PER_W
    sem_g = (sem_g0, sem_g1)
    sem_st = (sem_st0, sem_st1)
    sem_p = (sem_p0, sem_p1)

    def g_start(b, c, tb):
        pltpu.async_copy(
            tok_table_hbm.at[idx_v.at[b, c]], tok_v.at[tb], sem_g[tb])

    def g_wait(tb):
        pltpu.make_async_copy(
            tok_table_hbm.at[idx_v.at[0, 0]], tok_v.at[tb],
            sem_g[tb]).wait()

    def st_start(b, c, tb):
        pltpu.async_copy(
            tok_v.at[tb], out_hbm.at[pl.ds(b * S + p0 + c * R, R)],
            sem_st[tb])

    def st_wait(tb):
        pltpu.make_async_copy(
            tok_v.at[tb], out_hbm.at[pl.ds(p0, R)], sem_st[tb]).wait()

    def p_start(c, pb):
        pltpu.async_copy(
            pos_hbm.at[pl.ds(p0 + c * R, R)], pos_v.at[pb], sem_p[pb])

    def p_wait(pb):
        pltpu.make_async_copy(
            pos_hbm.at[pl.ds(p0, R)], pos_v.at[pb], sem_p[pb]).wait()

    def add_rows(tb, pb):
        tv = tok_v.at[tb]
        pv = pos_v.at[pb]

        @functools.partial(plsc.parallel_loop, 0, R, unroll=2)
        def _(r):
            for k in range(DV):
                sl = pl.ds(k * LANES, LANES)
                plsc.addupdate(tv.at[r, sl], pv[r, sl])

    for b in range(B):
        pltpu.async_copy(ids_hbm.at[b, wid], idx_v.at[b], sem_p0)
    for b in range(B):
        pltpu.make_async_copy(
            ids_hbm.at[b, wid], idx_v.at[b], sem_p0).wait()
    p_start(0, 0)
    g_start(0, 0, 0)

    def pair_body(c2, _):
        for rr in range(8):
            b = rr % 4
            tb = rr % 2
            pb = rr // 4
            c = 2 * c2 + pb
            if b == 0:
                p_wait(pb)
                if rr == 0:
                    p_start(c + 1, 1 - pb)
                else:
                    @pl.when(c2 < C2 - 1)
                    def _():
                        p_start(c + 1, 1 - pb)
            if rr == 0:
                @pl.when(c2 > 0)
                def _():
                    st_wait(1 - tb)
                g_start(b + 1, c, 1 - tb)
            elif rr < 7:
                st_wait(1 - tb)
                g_start((rr + 1) % 4, 2 * c2 + (rr + 1) // 4, 1 - tb)
            else:
                @pl.when(c2 < C2 - 1)
                def _():
                    st_wait(1 - tb)
                    g_start(0, 2 * c2 + 2, 1 - tb)
            g_wait(tb)
            add_rows(tb, pb)
            st_start(b, c, tb)
        return 0

    lax.fori_loop(0, C2, pair_body, 0, unroll=False)
    st_wait(0)
    st_wait(1)


@jax.jit
def _emb(ids, token_table, position_table):
    mesh = plsc.VectorSubcoreMesh(
        core_axis_name="c", subcore_axis_name="s", num_cores=NC,
        num_subcores=NS)
    run = pl.kernel(
        _emb_kernel,
        out_type=jax.ShapeDtypeStruct((B * S, D), jnp.float32),
        mesh=mesh,
        scratch_types=[
            pltpu.VMEM((B, C, R), jnp.int32),
            pltpu.VMEM((2, R, D), jnp.float32),
            pltpu.VMEM((2, R, D), jnp.float32),
            pltpu.SemaphoreType.DMA,
            pltpu.SemaphoreType.DMA,
            pltpu.SemaphoreType.DMA,
            pltpu.SemaphoreType.DMA,
            pltpu.SemaphoreType.DMA,
            pltpu.SemaphoreType.DMA,
        ],
    )
    return run(ids, token_table, position_table)


def kernel(token_ids, token_table, position_table):
    ids = token_ids.astype(jnp.int32)
    ids_w = ids.reshape(B, NW, C, R)
    out = _emb(ids_w, token_table, position_table)
    return out.reshape(B, S, D)
